# baseline (speedup 1.0000x reference)
import functools

import jax
import jax.numpy as jnp
from jax import lax
from jax.experimental import pallas as pl
from jax.experimental.pallas import tpu as pltpu
from jax.experimental.pallas import tpu_sc as plsc

NUM_Q = 900
HIDDEN = 256
HALF = 128
B_TOTAL = 16384 * 20
NUM_PAIRS = 16
B_PER_P = B_TOTAL // NUM_PAIRS
CHUNK = 40
NCHUNK = B_PER_P // CHUNK

_mesh = plsc.VectorSubcoreMesh(core_axis_name="c", subcore_axis_name="s")


@functools.partial(
    pl.kernel,
    mesh=_mesh,
    out_type=jax.ShapeDtypeStruct((B_TOTAL, HIDDEN), jnp.float32),
    compiler_params=pltpu.CompilerParams(needs_layout_passes=False),
    scratch_types=[
        pltpu.VMEM((NUM_Q * HALF,), jnp.float32),
        pltpu.VMEM((2, CHUNK, HALF), jnp.float32),
        pltpu.VMEM((2 * CHUNK,), jnp.int32),
        pltpu.SemaphoreType.DMA,
        pltpu.SemaphoreType.DMA,
    ],
)
def _embed_gather(tbl0_hbm, tbl1_hbm, idx_hbm, out_hbm, tbl_v, outb, idxb,
                  isem, wsem):
    cid = lax.axis_index("c")
    sid = lax.axis_index("s")
    rowbase = sid * B_PER_P
    colbase = cid * HALF

    @pl.when(cid == 0)
    def _():
        pltpu.sync_copy(tbl0_hbm, tbl_v)

    @pl.when(cid == 1)
    def _():
        pltpu.sync_copy(tbl1_hbm, tbl_v)

    lane = jnp.arange(16, dtype=jnp.int32)

    def start_idx_copy(c, slot):
        pltpu.async_copy(
            idx_hbm.at[pl.ds(rowbase + c * CHUNK, CHUNK)],
            idxb.at[pl.ds(slot * CHUNK, CHUNK)],
            isem,
        )

    def wait_idx():
        pltpu.make_async_copy(
            idx_hbm.at[pl.ds(0, CHUNK)], idxb.at[pl.ds(0, CHUNK)], isem
        ).wait()

    def wait_write():
        pltpu.make_async_copy(
            outb.at[0], out_hbm.at[pl.ds(0, CHUNK), pl.ds(0, HALF)], wsem
        ).wait()

    start_idx_copy(0, 0)

    def chunk_body(c, carry):
        p = lax.rem(c, 2)
        wait_idx()

        @pl.when(c + 1 < NCHUNK)
        def _():
            start_idx_copy(c + 1, 1 - p)

        @pl.when(c >= 2)
        def _():
            wait_write()

        def row_body(i, carry2):
            rsplat = plsc.load_gather(
                idxb, [jnp.full((16,), p * CHUNK + i, jnp.int32)]
            )
            raddr = rsplat * HALF
            for k in range(HALF // 16):
                vals = plsc.load_gather(tbl_v, [raddr + (k * 16 + lane)])
                outb[p, i, pl.ds(k * 16, 16)] = vals
            return carry2

        lax.fori_loop(0, CHUNK, row_body, 0)

        pltpu.async_copy(
            outb.at[p],
            out_hbm.at[pl.ds(rowbase + c * CHUNK, CHUNK), pl.ds(colbase, HALF)],
            wsem,
        )
        return carry

    lax.fori_loop(0, NCHUNK, chunk_body, 0)
    wait_write()
    wait_write()


def kernel(indices, query_embed_weight):
    idx = indices.reshape(-1).astype(jnp.int32)
    tbl0 = query_embed_weight[:, :HALF].reshape(-1)
    tbl1 = query_embed_weight[:, HALF:].reshape(-1)
    out = _embed_gather(tbl0, tbl1, idx)
    return out.reshape(indices.shape + (HIDDEN,))


# speedup vs baseline: 1.1784x; 1.1784x over previous
import functools

import jax
import jax.numpy as jnp
from jax import lax
from jax.experimental import pallas as pl
from jax.experimental.pallas import tpu as pltpu
from jax.experimental.pallas import tpu_sc as plsc

NUM_Q = 900
HIDDEN = 256
HALF = 128
B_TOTAL = 16384 * 20
NUM_PAIRS = 16
B_PER_P = B_TOTAL // NUM_PAIRS
CHUNK = 32
NCHUNK = B_PER_P // CHUNK

_mesh = plsc.VectorSubcoreMesh(core_axis_name="c", subcore_axis_name="s")
_SPLAT_DNUMS = lax.GatherDimensionNumbers(
    offset_dims=(), collapsed_slice_dims=(0,), start_index_map=(0,)
)


@functools.partial(
    pl.kernel,
    mesh=_mesh,
    out_type=jax.ShapeDtypeStruct((B_TOTAL, HIDDEN), jnp.float32),
    compiler_params=pltpu.CompilerParams(needs_layout_passes=False),
    scratch_types=[
        pltpu.VMEM((NUM_Q * HALF,), jnp.float32),
        pltpu.VMEM((2, CHUNK, HALF), jnp.float32),
        pltpu.VMEM((2 * CHUNK,), jnp.int32),
        pltpu.SemaphoreType.DMA,
        pltpu.SemaphoreType.DMA,
    ],
)
def _embed_gather(tbl0_hbm, tbl1_hbm, idx_hbm, out_hbm, tbl_v, outb, idxb,
                  isem, wsem):
    cid = lax.axis_index("c")
    sid = lax.axis_index("s")
    rowbase = sid * B_PER_P
    colbase = cid * HALF

    @pl.when(cid == 0)
    def _():
        pltpu.sync_copy(tbl0_hbm, tbl_v)

    @pl.when(cid == 1)
    def _():
        pltpu.sync_copy(tbl1_hbm, tbl_v)

    lane = jnp.arange(16, dtype=jnp.int32)

    def start_idx_copy(c, slot):
        pltpu.async_copy(
            idx_hbm.at[pl.ds(rowbase + c * CHUNK, CHUNK)],
            idxb.at[pl.ds(slot * CHUNK, CHUNK)],
            isem,
        )

    def wait_idx():
        pltpu.make_async_copy(
            idx_hbm.at[pl.ds(0, CHUNK)], idxb.at[pl.ds(0, CHUNK)], isem
        ).wait()

    def wait_write():
        pltpu.make_async_copy(
            outb.at[0], out_hbm.at[pl.ds(0, CHUNK), pl.ds(0, HALF)], wsem
        ).wait()

    start_idx_copy(0, 0)

    def chunk_body(c, carry):
        p = lax.rem(c, 2)
        wait_idx()

        @pl.when(c + 1 < NCHUNK)
        def _():
            start_idx_copy(c + 1, 1 - p)

        @pl.when(c >= 2)
        def _():
            wait_write()

        def batch_body(rb, carry2):
            idx16 = idxb[pl.ds(p * CHUNK + rb * 16, 16)]
            addr16 = idx16 * HALF
            for i in range(16):
                raddr = lax.gather(
                    addr16,
                    jnp.full((16, 1), i, jnp.int32),
                    _SPLAT_DNUMS,
                    slice_sizes=(1,),
                    mode=lax.GatherScatterMode.PROMISE_IN_BOUNDS,
                )
                for k in range(HALF // 16):
                    vals = plsc.load_gather(tbl_v, [raddr + (k * 16 + lane)])
                    outb[p, rb * 16 + i, pl.ds(k * 16, 16)] = vals
            return carry2

        lax.fori_loop(0, CHUNK // 16, batch_body, 0)

        pltpu.async_copy(
            outb.at[p],
            out_hbm.at[pl.ds(rowbase + c * CHUNK, CHUNK), pl.ds(colbase, HALF)],
            wsem,
        )
        return carry

    lax.fori_loop(0, NCHUNK, chunk_body, 0)
    wait_write()
    wait_write()


def kernel(indices, query_embed_weight):
    idx = indices.reshape(-1).astype(jnp.int32)
    tbl0 = query_embed_weight[:, :HALF].reshape(-1)
    tbl1 = query_embed_weight[:, HALF:].reshape(-1)
    out = _embed_gather(tbl0, tbl1, idx)
    return out.reshape(indices.shape + (HIDDEN,))
